# baseline (speedup 1.0000x reference)
import math

import jax
import jax.numpy as jnp
from jax import lax
from jax.experimental import pallas as pl
from jax.experimental.pallas import tpu as pltpu
from jax.experimental.pallas import tpu_sc as plsc

N = 10000
DEG = 16
E = N * DEG
R = 2
B = 2
C_IN = 3
T_IN = 12
D_IN = C_IN * T_IN
D = 64
H = 4
DK = D // H
L = 2
T_OUT = 12

NCORE = 2
NSUB = 16
NT = NCORE * NSUB
NB = 320
NPAD = NT * NB
FLUSH = 2048
ECAP = E + 8192
CHB = 4000
CH = 128
LV = 6144
LVCAP = ((E + LV - 1) // LV) * LV

_MESH = plsc.VectorSubcoreMesh(core_axis_name="c", subcore_axis_name="s")


def _bin_body(src_hbm, dst_hbm, lp0, lp1, cnts,
              stage_s, stage_d, buf, cnt_v):
    wid = lax.axis_index("c") * NSUB + lax.axis_index("s")
    base = wid * NB
    iota = lax.iota(jnp.int32, 16)
    zero16 = jnp.zeros((16,), jnp.int32)

    def zinit(i, _):
        buf[pl.ds(i * 16, 16)] = zero16
        return 0

    lax.fori_loop(0, (FLUSH + 16) // 16, zinit, 0)

    for r, lpr in enumerate((lp0, lp1)):
        def outer(cb, carry, lpr=lpr, r=r):
            cnt, written = carry
            off = pl.multiple_of(r * E + cb * CHB, 8)
            pltpu.sync_copy(src_hbm.at[pl.ds(off, CHB)], stage_s)
            pltpu.sync_copy(dst_hbm.at[pl.ds(off, CHB)], stage_d)

            def inner(i, carry2):
                cnt, written = carry2
                d16 = stage_d[pl.ds(i * 16, 16)]
                s16 = stage_s[pl.ds(i * 16, 16)]
                dl = d16 - base
                msk = (dl >= 0) & (dl < NB)
                dlc = jnp.clip(dl, 0, NB - 1)
                packed = s16 * 512 + dlc
                key = jnp.where(msk, iota, jnp.full((16,), 16, jnp.int32))
                _, sv = lax.sort((key, packed), num_keys=1)
                buf[pl.ds(cnt, 16)] = sv
                cnt = cnt + plsc.all_reduce_population_count(msk)[0]

                def do_flush(c, w):
                    wo = pl.multiple_of(wid * ECAP + w, 8)
                    pltpu.sync_copy(buf.at[pl.ds(0, FLUSH)],
                                    lpr.at[pl.ds(wo, FLUSH)])
                    rem = buf[pl.ds(FLUSH, 16)]
                    buf[pl.ds(0, 16)] = rem
                    return c - FLUSH, w + FLUSH

                cnt, written = lax.cond(cnt >= FLUSH, do_flush,
                                        lambda c, w: (c, w), cnt, written)
                return (cnt, written)

            return lax.fori_loop(0, CHB // 16, inner, (cnt, written))

        cnt, written = lax.fori_loop(0, E // CHB, outer,
                                     (jnp.int32(0), jnp.int32(0)))
        wo = pl.multiple_of(wid * ECAP + written, 8)
        pltpu.sync_copy(buf.at[pl.ds(0, FLUSH)], lpr.at[pl.ds(wo, FLUSH)])
        cnt_v[...] = jnp.broadcast_to(written + cnt, (16,)).astype(jnp.int32)
        co = pl.multiple_of((r * NT + wid) * 16, 8)
        pltpu.sync_copy(cnt_v, cnts.at[pl.ds(co, 16)])


def _bin_edges(src, dst):
    f = pl.kernel(
        _bin_body,
        out_type=(
            jax.ShapeDtypeStruct((NT * ECAP,), jnp.int32),
            jax.ShapeDtypeStruct((NT * ECAP,), jnp.int32),
            jax.ShapeDtypeStruct((R * NT * 16,), jnp.int32),
        ),
        mesh=_MESH,
        compiler_params=pltpu.CompilerParams(needs_layout_passes=False),
        scratch_types=[
            pltpu.VMEM((CHB,), jnp.int32),
            pltpu.VMEM((CHB,), jnp.int32),
            pltpu.VMEM((FLUSH + 16,), jnp.int32),
            pltpu.VMEM((16,), jnp.int32),
        ],
    )
    return f(src.reshape(R * E), dst.reshape(R * E))


def _edge_body(q_hbm, kr0, kr1, vr0, vr1, lp0, lp1, cnts,
               agg0, agg1, lg,
               q_loc, m_buf, s_buf, acc, pk_full, tl_sup,
               src_v0, src_v1, dl_v0, dl_v1, rows0, rows1,
               cnt_v, sem0, sem1):
    wid = lax.axis_index("c") * NSUB + lax.axis_index("s")
    base = wid * NB
    iota = lax.iota(jnp.int32, 16)
    iota_h = jnp.minimum(iota, 3)
    zf = jnp.zeros((16,), jnp.float32)
    minf = jnp.full((16,), -3.0e38, jnp.float32)
    srcs = (src_v0, src_v1)
    dls = (dl_v0, dl_v1)
    rows = (rows0, rows1)
    sems = (sem0, sem1)

    pltpu.sync_copy(q_hbm.at[pl.ds(pl.multiple_of(base * D, 8), NB * D)],
                    q_loc)

    def unpack_chunk(ci, p):
        o = ci * CH

        def u(ui, _):
            pv = pk_full[pl.ds(o + ui * 16, 16)]
            sv = jnp.clip(jnp.right_shift(pv, 9), 0, N - 1)
            dv = jnp.minimum(jnp.bitwise_and(pv, 511), NB - 1)
            srcs[p][pl.ds(ui * 16, 16)] = sv
            dls[p][pl.ds(ui * 16, 16)] = dv
            return 0

        lax.fori_loop(0, CH // 16, u, 0)

    for r in range(R):
        krr = (kr0, kr1)[r]
        vrr = (vr0, vr1)[r]
        lpr = (lp0, lp1)[r]
        aggr = (agg0, agg1)[r]

        co = pl.multiple_of((r * NT + wid) * 16, 8)
        pltpu.sync_copy(cnts.at[pl.ds(co, 16)], cnt_v)
        cnt = cnt_v[...][0]

        for hh in range(4):
            m_buf[pl.ds(hh * 16, 16)] = minf

        def init_ms(i, _):
            s_buf[pl.ds(i * 16, 16)] = zf
            return 0

        lax.fori_loop(0, NB, init_ms, 0)

        def init_acc(i, _):
            acc[pl.ds(i * 16, 16)] = zf
            return 0

        lax.fori_loop(0, NB * 4, init_acc, 0)

        nsup = (cnt + LV - 1) // LV
        multi = nsup > 1

        def sup1(si, _, lpr=lpr, krr=krr, cnt=cnt, multi=multi):
            soff = si * LV
            lo = pl.multiple_of(wid * ECAP + soff, 8)
            pltpu.sync_copy(lpr.at[pl.ds(lo, LV)], pk_full)
            nval = jnp.minimum(cnt - soff, LV)
            nin = (nval + CH - 1) // CH

            @pl.when(nin > 0)
            def _():
                unpack_chunk(0, 0)
                pltpu.async_copy(krr.at[srcs[0]], rows[0], sems[0])

            def pair(ci2, _):
                for p_ in (0, 1):
                    ci = ci2 * 2 + p_

                    @pl.when(ci < nin)
                    def _(ci=ci, p_=p_):
                        @pl.when(ci + 1 < nin)
                        def _():
                            unpack_chunk(ci + 1, 1 - p_)
                            pltpu.async_copy(krr.at[srcs[1 - p_]],
                                             rows[1 - p_], sems[1 - p_])

                        pltpu.make_async_copy(krr.at[srcs[p_]], rows[p_],
                                              sems[p_]).wait()
                        coff = ci * CH

                        def sub(sc, mcar):
                            e0 = sc * 16
                            dl16 = dls[p_][pl.ds(e0, 16)]
                            dlb = dl16 * D
                            t = [zf, zf, zf, zf]
                            for c in range(D):
                                csp = jnp.full((16,), c, jnp.int32)
                                qv = plsc.load_gather(q_loc, [dlb + csp])
                                kv = plsc.load_gather(rows[p_],
                                                      [iota + e0, csp])
                                t[c // 16] = t[c // 16] + qv * kv
                            vvalid = (soff + coff + e0 + iota) < cnt
                            mcar = list(mcar)
                            for hh in range(4):
                                tl_sup[pl.ds(hh * LV + coff + e0, 16)] = t[hh]
                                mcar[hh] = jnp.maximum(
                                    mcar[hh], jnp.where(vvalid, t[hh], minf))
                            return tuple(mcar)

                        mcar = lax.fori_loop(0, CH // 16, sub,
                                             tuple(m_buf[pl.ds(hh * 16, 16)]
                                                   for hh in range(4)))
                        for hh in range(4):
                            m_buf[pl.ds(hh * 16, 16)] = mcar[hh]

                return 0

            lax.fori_loop(0, (nin + 1) // 2, pair, 0)

            @pl.when(multi)
            def _():
                go = pl.multiple_of((wid * LVCAP + soff) * 4, 8)
                pltpu.sync_copy(tl_sup, lg.at[pl.ds(go, 4 * LV)])

            return 0

        lax.fori_loop(0, nsup, sup1, 0)

        gmax = [jnp.max(m_buf[pl.ds(hh * 16, 16)]) for hh in range(4)]

        def sup2(si, _, lpr=lpr, vrr=vrr, cnt=cnt, multi=multi, gmax=gmax):
            soff = si * LV
            lo = pl.multiple_of(wid * ECAP + soff, 8)
            pltpu.sync_copy(lpr.at[pl.ds(lo, LV)], pk_full)

            @pl.when(multi)
            def _():
                go = pl.multiple_of((wid * LVCAP + soff) * 4, 8)
                pltpu.sync_copy(lg.at[pl.ds(go, 4 * LV)], tl_sup)

            nval = jnp.minimum(cnt - soff, LV)
            nin = (nval + CH - 1) // CH

            @pl.when(nin > 0)
            def _():
                unpack_chunk(0, 0)
                pltpu.async_copy(vrr.at[srcs[0]], rows[0], sems[0])

            def pair(ci2, _):
                for p_ in (0, 1):
                    ci = ci2 * 2 + p_

                    @pl.when(ci < nin)
                    def _(ci=ci, p_=p_):
                        @pl.when(ci + 1 < nin)
                        def _():
                            unpack_chunk(ci + 1, 1 - p_)
                            pltpu.async_copy(vrr.at[srcs[1 - p_]],
                                             rows[1 - p_], sems[1 - p_])

                        pltpu.make_async_copy(vrr.at[srcs[p_]], rows[p_],
                                              sems[p_]).wait()
                        coff = ci * CH

                        def sub2(sc, _):
                            e0 = sc * 16
                            dl16 = dls[p_][pl.ds(e0, 16)]
                            dlm = dl16 * 16
                            dlb = dl16 * D
                            vvalid = (soff + coff + e0 + iota) < cnt
                            ev = []
                            for hh in range(4):
                                tv = tl_sup[pl.ds(hh * LV + coff + e0, 16)]
                                e_ = jnp.exp(tv - gmax[hh])
                                ev.append(jnp.where(vvalid, e_, zf))
                            for hh in range(4):
                                hsp = jnp.full((16,), hh, jnp.int32)
                                plsc.addupdate_scatter(
                                    s_buf, [dlm + hsp], ev[hh])
                            for c in range(D):
                                csp = jnp.full((16,), c, jnp.int32)
                                vv = plsc.load_gather(rows[p_],
                                                      [iota + e0, csp])
                                plsc.addupdate_scatter(
                                    acc, [dlb + csp], vv * ev[c // 16])
                            return 0

                        lax.fori_loop(0, CH // 16, sub2, 0)

                return 0

            lax.fori_loop(0, (nin + 1) // 2, pair, 0)
            return 0

        lax.fori_loop(0, nsup, sup2, 0)

        def fin(i, _):
            srow = s_buf[pl.ds(i * 16, 16)]
            for hh in range(4):
                sv = srow[hh]
                sv = jnp.where(sv > 0.0, sv, 1.0)
                o = pl.ds(i * D + hh * 16, 16)
                acc[o] = acc[o] / sv
            return 0

        lax.fori_loop(0, NB, fin, 0)
        pltpu.sync_copy(
            acc, aggr.at[pl.ds(pl.multiple_of(base * D, 8), NB * D)])


def _edge_pass(qflat, kr0, kr1, vr0, vr1, lp0, lp1, cnts):
    f = pl.kernel(
        _edge_body,
        out_type=(
            jax.ShapeDtypeStruct((NPAD * D,), jnp.float32),
            jax.ShapeDtypeStruct((NPAD * D,), jnp.float32),
            jax.ShapeDtypeStruct((NT * H * LVCAP,), jnp.float32),
        ),
        mesh=_MESH,
        compiler_params=pltpu.CompilerParams(needs_layout_passes=False),
        scratch_types=[
            pltpu.VMEM((NB * D,), jnp.float32),
            pltpu.VMEM((64,), jnp.float32),
            pltpu.VMEM((NB * 16,), jnp.float32),
            pltpu.VMEM((NB * D,), jnp.float32),
            pltpu.VMEM((LV,), jnp.int32),
            pltpu.VMEM((H * LV,), jnp.float32),
            pltpu.VMEM((CH,), jnp.int32),
            pltpu.VMEM((CH,), jnp.int32),
            pltpu.VMEM((CH,), jnp.int32),
            pltpu.VMEM((CH,), jnp.int32),
            pltpu.VMEM((CH, 2 * D), jnp.float32),
            pltpu.VMEM((CH, 2 * D), jnp.float32),
            pltpu.VMEM((16,), jnp.int32),
            pltpu.SemaphoreType.DMA,
            pltpu.SemaphoreType.DMA,
        ],
    )
    a0, a1, _ = f(qflat, kr0, kr1, vr0, vr1, lp0, lp1, cnts)
    return a0.reshape(NPAD, D), a1.reshape(NPAD, D)


def _pre_body(x_ref, w_ref, b_ref, o_ref):
    h = jnp.dot(x_ref[...], w_ref[...],
                preferred_element_type=jnp.float32) + b_ref[...]
    o_ref[...] = jax.nn.gelu(h)


def _q_mm_body(h_ref, w_ref, b_ref, o_ref):
    o_ref[...] = jnp.dot(h_ref[...], w_ref[...],
                         preferred_element_type=jnp.float32) + b_ref[...]


def _q_mm(hp, w, b):
    return pl.pallas_call(
        _q_mm_body,
        out_shape=jax.ShapeDtypeStruct((NPAD, D), jnp.float32),
    )(hp, w, b.reshape(1, D))


def _pre(xp, w, b):
    return pl.pallas_call(
        _pre_body,
        out_shape=jax.ShapeDtypeStruct((NPAD, D), jnp.float32),
    )(xp, w, b.reshape(1, D))


def _qkv_body(h_ref, w_ref, b_ref, o_ref):
    res = jnp.dot(h_ref[...], w_ref[0],
                  preferred_element_type=jnp.float32) + b_ref[0]
    o_ref[0] = jnp.concatenate([res, jnp.zeros_like(res)], axis=-1)


def _qkv(hp, wstack, bstack):
    nmat = wstack.shape[0]
    bstack = bstack.reshape(nmat, 1, D)
    return pl.pallas_call(
        _qkv_body,
        grid=(nmat,),
        in_specs=[
            pl.BlockSpec((NPAD, D), lambda i: (0, 0)),
            pl.BlockSpec((1, D, D), lambda i: (i, 0, 0)),
            pl.BlockSpec((1, 1, D), lambda i: (i, 0, 0)),
        ],
        out_specs=pl.BlockSpec((1, NPAD, 2 * D), lambda i: (i, 0, 0)),
        out_shape=jax.ShapeDtypeStruct((nmat, NPAD, 2 * D), jnp.float32),
    )(hp, wstack, bstack)


def _combine_body(a0_ref, a1_ref, h_ref, wa_ref, ba_ref, g_ref, bb_ref,
                  skip_ref, o_ref):
    hs = (a0_ref[...] + a1_ref[...]) * 0.5
    trans = jnp.dot(hs, wa_ref[...],
                    preferred_element_type=jnp.float32) + ba_ref[...]
    alpha = jax.nn.sigmoid(skip_ref[0, 0])
    out = alpha * trans + (1.0 - alpha) * h_ref[...]
    mu = jnp.mean(out, axis=-1, keepdims=True)
    d = out - mu
    var = jnp.mean(d * d, axis=-1, keepdims=True)
    o_ref[...] = d * lax.rsqrt(var + 1e-5) * g_ref[...] + bb_ref[...]


def _combine(a0, a1, hp, wa, ba, g, bb, skip):
    return pl.pallas_call(
        _combine_body,
        out_shape=jax.ShapeDtypeStruct((NPAD, D), jnp.float32),
    )(a0, a1, hp, wa, ba.reshape(1, D), g.reshape(1, D), bb.reshape(1, D),
      skip.reshape(1, 1))


def _final_body(h_ref, wp_ref, bp_ref, wh_ref, bh_ref, o_ref):
    t = jnp.dot(h_ref[...], wp_ref[...],
                preferred_element_type=jnp.float32) + bp_ref[...]
    o_ref[...] = jnp.dot(t, wh_ref[...],
                         preferred_element_type=jnp.float32) + bh_ref[...]


def _final(hp, wp, bp, wh, bh):
    return pl.pallas_call(
        _final_body,
        out_shape=jax.ShapeDtypeStruct((NPAD, T_OUT), jnp.float32),
    )(hp, wp, bp.reshape(1, D), wh, bh.reshape(1, T_OUT))


def _block_diag(w):
    out = jnp.zeros((D, D), jnp.float32)
    for hh in range(H):
        out = out.at[hh * DK:(hh + 1) * DK, hh * DK:(hh + 1) * DK].set(w[hh])
    return out


def kernel(data, timestamps, params, src, dst):
    x = data.reshape(B, N, D_IN)
    xp = jnp.pad(x, ((0, 0), (0, NPAD - N), (0, 0)))

    lp0, lp1, cnts = _bin_edges(src, dst)

    layer_w = []
    for l in range(L):
        p = params['layers'][l]
        ws, bs = [], []
        for r in range(R):
            bd = _block_diag(p['w_att'][r])
            scale = jnp.repeat(p['mu'][r], DK) / math.sqrt(DK)
            ws.append((p['Wk'] @ bd) * scale[None, :])
            bs.append((p['bk'] @ bd) * scale)
        for r in range(R):
            bd = _block_diag(p['w_msg'][r])
            ws.append(p['Wv'] @ bd)
            bs.append(p['bv'] @ bd)
        layer_w.append((jnp.stack(ws), jnp.stack(bs)))

    outs = []
    for b in range(B):
        hp = _pre(xp[b], params['adapt_W'], params['adapt_b'])
        for l in range(L):
            p = params['layers'][l]
            wstack, bstack = layer_w[l]
            proj = _qkv(hp, wstack, bstack)
            q = _q_mm(hp, p['Wq'], p['bq'])
            a0, a1 = _edge_pass(q.reshape(NPAD * D), proj[0],
                                proj[1], proj[2], proj[3], lp0, lp1, cnts)
            hp = _combine(a0, a1, hp, p['Wa'], p['ba'], p['ln_g'], p['ln_b'],
                          p['skip'])
        outs.append(_final(hp, params['pred_W'], params['pred_b'],
                           params['head_W'], params['head_b'])[:N])
    return jnp.stack(outs, 0)


# speedup vs baseline: 1.4188x; 1.4188x over previous
import math

import jax
import jax.numpy as jnp
from jax import lax
from jax.experimental import pallas as pl
from jax.experimental.pallas import tpu as pltpu
from jax.experimental.pallas import tpu_sc as plsc

N = 10000
DEG = 16
E = N * DEG
R = 2
B = 2
C_IN = 3
T_IN = 12
D_IN = C_IN * T_IN
D = 64
H = 4
DK = D // H
L = 2
T_OUT = 12

NCORE = 2
NSUB = 16
NT = NCORE * NSUB
NB = 320
NPAD = NT * NB
FLUSH = 2048
ECAP = E + 8192
CHB = 4000
CH = 128
LV = 6144
LVCAP = ((E + LV - 1) // LV) * LV

_MESH = plsc.VectorSubcoreMesh(core_axis_name="c", subcore_axis_name="s")


def _bin_body(src_hbm, dst_hbm, lp0, lp1, cnts,
              stage_s, stage_d, buf, cnt_v):
    wid = lax.axis_index("c") * NSUB + lax.axis_index("s")
    base = wid * NB
    iota = lax.iota(jnp.int32, 16)
    zero16 = jnp.zeros((16,), jnp.int32)

    def zinit(i, _):
        buf[pl.ds(i * 16, 16)] = zero16
        return 0

    lax.fori_loop(0, (FLUSH + 16) // 16, zinit, 0)

    for r, lpr in enumerate((lp0, lp1)):
        def outer(cb, carry, lpr=lpr, r=r):
            cnt, written = carry
            off = pl.multiple_of(r * E + cb * CHB, 8)
            pltpu.sync_copy(src_hbm.at[pl.ds(off, CHB)], stage_s)
            pltpu.sync_copy(dst_hbm.at[pl.ds(off, CHB)], stage_d)

            def inner(i, carry2):
                cnt, written = carry2
                d16 = stage_d[pl.ds(i * 16, 16)]
                s16 = stage_s[pl.ds(i * 16, 16)]
                dl = d16 - base
                msk = (dl >= 0) & (dl < NB)
                dlc = jnp.clip(dl, 0, NB - 1)
                packed = s16 * 512 + dlc
                key = jnp.where(msk, iota, jnp.full((16,), 16, jnp.int32))
                _, sv = lax.sort((key, packed), num_keys=1)
                buf[pl.ds(cnt, 16)] = sv
                cnt = cnt + plsc.all_reduce_population_count(msk)[0]

                def do_flush(c, w):
                    wo = pl.multiple_of(wid * ECAP + w, 8)
                    pltpu.sync_copy(buf.at[pl.ds(0, FLUSH)],
                                    lpr.at[pl.ds(wo, FLUSH)])
                    rem = buf[pl.ds(FLUSH, 16)]
                    buf[pl.ds(0, 16)] = rem
                    return c - FLUSH, w + FLUSH

                cnt, written = lax.cond(cnt >= FLUSH, do_flush,
                                        lambda c, w: (c, w), cnt, written)
                return (cnt, written)

            return lax.fori_loop(0, CHB // 16, inner, (cnt, written))

        cnt, written = lax.fori_loop(0, E // CHB, outer,
                                     (jnp.int32(0), jnp.int32(0)))
        wo = pl.multiple_of(wid * ECAP + written, 8)
        pltpu.sync_copy(buf.at[pl.ds(0, FLUSH)], lpr.at[pl.ds(wo, FLUSH)])
        cnt_v[...] = jnp.broadcast_to(written + cnt, (16,)).astype(jnp.int32)
        co = pl.multiple_of((r * NT + wid) * 16, 8)
        pltpu.sync_copy(cnt_v, cnts.at[pl.ds(co, 16)])


def _bin_edges(src, dst):
    f = pl.kernel(
        _bin_body,
        out_type=(
            jax.ShapeDtypeStruct((NT * ECAP,), jnp.int32),
            jax.ShapeDtypeStruct((NT * ECAP,), jnp.int32),
            jax.ShapeDtypeStruct((R * NT * 16,), jnp.int32),
        ),
        mesh=_MESH,
        compiler_params=pltpu.CompilerParams(needs_layout_passes=False),
        scratch_types=[
            pltpu.VMEM((CHB,), jnp.int32),
            pltpu.VMEM((CHB,), jnp.int32),
            pltpu.VMEM((FLUSH + 16,), jnp.int32),
            pltpu.VMEM((16,), jnp.int32),
        ],
    )
    return f(src.reshape(R * E), dst.reshape(R * E))


def _edge_body(q_hbm, kv0, kv1, lp0, lp1, cnts,
               agg0, agg1,
               q_loc, m_lane, s_buf, acc, pk_full,
               src_v0, src_v1, dl_v0, dl_v1, rows0, rows1,
               cnt_v, sem0, sem1):
    wid = lax.axis_index("c") * NSUB + lax.axis_index("s")
    base = wid * NB
    iota = lax.iota(jnp.int32, 16)
    zf = jnp.zeros((16,), jnp.float32)
    minf = jnp.full((16,), -3.0e38, jnp.float32)
    srcs = (src_v0, src_v1)
    dls = (dl_v0, dl_v1)
    rows = (rows0, rows1)
    sems = (sem0, sem1)

    pltpu.sync_copy(q_hbm.at[pl.ds(pl.multiple_of(base * D, 8), NB * D)],
                    q_loc)

    def unpack_chunk(ci, p):
        o = ci * CH

        def u(ui, _):
            pv = pk_full[pl.ds(o + ui * 16, 16)]
            sv = jnp.clip(jnp.right_shift(pv, 9), 0, N - 1)
            dv = jnp.minimum(jnp.bitwise_and(pv, 511), NB - 1)
            srcs[p][pl.ds(ui * 16, 16)] = sv
            dls[p][pl.ds(ui * 16, 16)] = dv
            return 0

        lax.fori_loop(0, CH // 16, u, 0)

    for r in range(R):
        kvr = (kv0, kv1)[r]
        lpr = (lp0, lp1)[r]
        aggr = (agg0, agg1)[r]

        co = pl.multiple_of((r * NT + wid) * 16, 8)
        pltpu.sync_copy(cnts.at[pl.ds(co, 16)], cnt_v)
        cnt = cnt_v[...][0]

        m_lane[...] = minf

        def init_ms(i, _):
            s_buf[pl.ds(i * 16, 16)] = zf
            return 0

        lax.fori_loop(0, NB, init_ms, 0)

        def init_acc(i, _):
            acc[pl.ds(i * 16, 16)] = zf
            return 0

        lax.fori_loop(0, NB * 4, init_acc, 0)

        nsup = (cnt + LV - 1) // LV

        def sup(si, _, lpr=lpr, kvr=kvr, cnt=cnt):
            soff = si * LV
            lo = pl.multiple_of(wid * ECAP + soff, 8)
            pltpu.sync_copy(lpr.at[pl.ds(lo, LV)], pk_full)
            nval = jnp.minimum(cnt - soff, LV)
            nin = (nval + CH - 1) // CH

            @pl.when(nin > 0)
            def _():
                unpack_chunk(0, 0)
                pltpu.async_copy(kvr.at[srcs[0]], rows[0], sems[0])

            def pair(ci2, _):
                for p_ in (0, 1):
                    ci = ci2 * 2 + p_

                    @pl.when(ci < nin)
                    def _(ci=ci, p_=p_):
                        @pl.when(ci + 1 < nin)
                        def _():
                            unpack_chunk(ci + 1, 1 - p_)
                            pltpu.async_copy(kvr.at[srcs[1 - p_]],
                                             rows[1 - p_], sems[1 - p_])

                        pltpu.make_async_copy(kvr.at[srcs[p_]], rows[p_],
                                              sems[p_]).wait()
                        coff = ci * CH

                        def sub(sc, _):
                            e0 = sc * 16
                            dl16 = dls[p_][pl.ds(e0, 16)]
                            dlb = dl16 * D
                            dlm = dl16 * 16
                            t = [zf, zf, zf, zf]
                            for c in range(D):
                                csp = jnp.full((16,), c, jnp.int32)
                                qv = plsc.load_gather(q_loc, [dlb + csp])
                                kv = plsc.load_gather(rows[p_],
                                                      [iota + e0, csp])
                                t[c // 16] = t[c // 16] + qv * kv
                            vvalid = (soff + coff + e0 + iota) < cnt
                            tm = [jnp.where(vvalid, t[hh], minf)
                                  for hh in range(4)]
                            m_old = m_lane[...]
                            cand = minf
                            for hh in range(4):
                                cand = jnp.where(iota == hh,
                                                 jnp.max(tm[hh]), cand)
                            m_new = jnp.maximum(m_old, cand)
                            m_lane[...] = m_new
                            scale = jnp.exp(m_old - m_new)

                            for hh in range(4):
                                sc_h = scale[hh]

                                @pl.when(sc_h < 1.0)
                                def _(sc_h=sc_h, hh=hh):
                                    def racc(i, _):
                                        o = pl.ds(i * D + hh * 16, 16)
                                        acc[o] = acc[o] * sc_h
                                        return 0

                                    lax.fori_loop(0, NB, racc, 0)

                            @pl.when(jnp.min(scale) < 1.0)
                            def _(scale=scale):
                                def rsb(i, _):
                                    o = pl.ds(i * 16, 16)
                                    s_buf[o] = s_buf[o] * scale
                                    return 0

                                lax.fori_loop(0, NB, rsb, 0)

                            ev = []
                            for hh in range(4):
                                e_ = jnp.exp(tm[hh] - m_new[hh])
                                ev.append(jnp.where(vvalid, e_, zf))
                            for hh in range(4):
                                hsp = jnp.full((16,), hh, jnp.int32)
                                plsc.addupdate_scatter(
                                    s_buf, [dlm + hsp], ev[hh])
                            for j in range(16):
                                dlj = dl16[j]
                                for hh in range(4):
                                    e_h = ev[hh][j]
                                    vrv = rows[p_][e0 + j,
                                                   pl.ds(D + hh * 16, 16)]
                                    plsc.addupdate(
                                        acc.at[pl.ds(dlj * D + hh * 16, 16)],
                                        vrv * e_h)
                            return 0

                        lax.fori_loop(0, CH // 16, sub, 0)

                return 0

            lax.fori_loop(0, (nin + 1) // 2, pair, 0)
            return 0

        lax.fori_loop(0, nsup, sup, 0)

        def fin(i, _):
            srow = s_buf[pl.ds(i * 16, 16)]
            for hh in range(4):
                sv = srow[hh]
                sv = jnp.where(sv > 0.0, sv, 1.0)
                o = pl.ds(i * D + hh * 16, 16)
                acc[o] = acc[o] / sv
            return 0

        lax.fori_loop(0, NB, fin, 0)
        pltpu.sync_copy(
            acc, aggr.at[pl.ds(pl.multiple_of(base * D, 8), NB * D)])


def _edge_pass(qflat, kv0, kv1, lp0, lp1, cnts):
    f = pl.kernel(
        _edge_body,
        out_type=(
            jax.ShapeDtypeStruct((NPAD * D,), jnp.float32),
            jax.ShapeDtypeStruct((NPAD * D,), jnp.float32),
        ),
        mesh=_MESH,
        compiler_params=pltpu.CompilerParams(needs_layout_passes=False),
        scratch_types=[
            pltpu.VMEM((NB * D,), jnp.float32),
            pltpu.VMEM((16,), jnp.float32),
            pltpu.VMEM((NB * 16,), jnp.float32),
            pltpu.VMEM((NB * D,), jnp.float32),
            pltpu.VMEM((LV,), jnp.int32),
            pltpu.VMEM((CH,), jnp.int32),
            pltpu.VMEM((CH,), jnp.int32),
            pltpu.VMEM((CH,), jnp.int32),
            pltpu.VMEM((CH,), jnp.int32),
            pltpu.VMEM((CH, 2 * D), jnp.float32),
            pltpu.VMEM((CH, 2 * D), jnp.float32),
            pltpu.VMEM((16,), jnp.int32),
            pltpu.SemaphoreType.DMA,
            pltpu.SemaphoreType.DMA,
        ],
    )
    a0, a1 = f(qflat, kv0, kv1, lp0, lp1, cnts)
    return a0.reshape(NPAD, D), a1.reshape(NPAD, D)


def _pre_body(x_ref, w_ref, b_ref, o_ref):
    h = jnp.dot(x_ref[...], w_ref[...],
                preferred_element_type=jnp.float32) + b_ref[...]
    o_ref[...] = jax.nn.gelu(h)


def _q_mm_body(h_ref, w_ref, b_ref, o_ref):
    o_ref[...] = jnp.dot(h_ref[...], w_ref[...],
                         preferred_element_type=jnp.float32) + b_ref[...]


def _q_mm(hp, w, b):
    return pl.pallas_call(
        _q_mm_body,
        out_shape=jax.ShapeDtypeStruct((NPAD, D), jnp.float32),
    )(hp, w, b.reshape(1, D))


def _pre(xp, w, b):
    return pl.pallas_call(
        _pre_body,
        out_shape=jax.ShapeDtypeStruct((NPAD, D), jnp.float32),
    )(xp, w, b.reshape(1, D))


def _qkv_body(h_ref, w_ref, b_ref, o_ref):
    o_ref[0] = jnp.dot(h_ref[...], w_ref[0],
                       preferred_element_type=jnp.float32) + b_ref[0]


def _qkv(hp, wstack, bstack):
    nmat = wstack.shape[0]
    bstack = bstack.reshape(nmat, 1, 2 * D)
    return pl.pallas_call(
        _qkv_body,
        grid=(nmat,),
        in_specs=[
            pl.BlockSpec((NPAD, D), lambda i: (0, 0)),
            pl.BlockSpec((1, D, 2 * D), lambda i: (i, 0, 0)),
            pl.BlockSpec((1, 1, 2 * D), lambda i: (i, 0, 0)),
        ],
        out_specs=pl.BlockSpec((1, NPAD, 2 * D), lambda i: (i, 0, 0)),
        out_shape=jax.ShapeDtypeStruct((nmat, NPAD, 2 * D), jnp.float32),
    )(hp, wstack, bstack)


def _combine_body(a0_ref, a1_ref, h_ref, wa_ref, ba_ref, g_ref, bb_ref,
                  skip_ref, o_ref):
    hs = (a0_ref[...] + a1_ref[...]) * 0.5
    trans = jnp.dot(hs, wa_ref[...],
                    preferred_element_type=jnp.float32) + ba_ref[...]
    alpha = jax.nn.sigmoid(skip_ref[0, 0])
    out = alpha * trans + (1.0 - alpha) * h_ref[...]
    mu = jnp.mean(out, axis=-1, keepdims=True)
    d = out - mu
    var = jnp.mean(d * d, axis=-1, keepdims=True)
    o_ref[...] = d * lax.rsqrt(var + 1e-5) * g_ref[...] + bb_ref[...]


def _combine(a0, a1, hp, wa, ba, g, bb, skip):
    return pl.pallas_call(
        _combine_body,
        out_shape=jax.ShapeDtypeStruct((NPAD, D), jnp.float32),
    )(a0, a1, hp, wa, ba.reshape(1, D), g.reshape(1, D), bb.reshape(1, D),
      skip.reshape(1, 1))


def _final_body(h_ref, wp_ref, bp_ref, wh_ref, bh_ref, o_ref):
    t = jnp.dot(h_ref[...], wp_ref[...],
                preferred_element_type=jnp.float32) + bp_ref[...]
    o_ref[...] = jnp.dot(t, wh_ref[...],
                         preferred_element_type=jnp.float32) + bh_ref[...]


def _final(hp, wp, bp, wh, bh):
    return pl.pallas_call(
        _final_body,
        out_shape=jax.ShapeDtypeStruct((NPAD, T_OUT), jnp.float32),
    )(hp, wp, bp.reshape(1, D), wh, bh.reshape(1, T_OUT))


def _block_diag(w):
    out = jnp.zeros((D, D), jnp.float32)
    for hh in range(H):
        out = out.at[hh * DK:(hh + 1) * DK, hh * DK:(hh + 1) * DK].set(w[hh])
    return out


def kernel(data, timestamps, params, src, dst):
    x = data.reshape(B, N, D_IN)
    xp = jnp.pad(x, ((0, 0), (0, NPAD - N), (0, 0)))

    lp0, lp1, cnts = _bin_edges(src, dst)

    layer_w = []
    for l in range(L):
        p = params['layers'][l]
        ws, bs = [], []
        for r in range(R):
            bda = _block_diag(p['w_att'][r])
            scale = jnp.repeat(p['mu'][r], DK) / math.sqrt(DK)
            wk = (p['Wk'] @ bda) * scale[None, :]
            bk = (p['bk'] @ bda) * scale
            bdm = _block_diag(p['w_msg'][r])
            wv = p['Wv'] @ bdm
            bv = p['bv'] @ bdm
            ws.append(jnp.concatenate([wk, wv], axis=1))
            bs.append(jnp.concatenate([bk, bv]))
        layer_w.append((jnp.stack(ws), jnp.stack(bs)))

    outs = []
    for b in range(B):
        hp = _pre(xp[b], params['adapt_W'], params['adapt_b'])
        for l in range(L):
            p = params['layers'][l]
            wstack, bstack = layer_w[l]
            proj = _qkv(hp, wstack, bstack)
            q = _q_mm(hp, p['Wq'], p['bq'])
            a0, a1 = _edge_pass(q.reshape(NPAD * D), proj[0],
                                proj[1], lp0, lp1, cnts)
            hp = _combine(a0, a1, hp, p['Wa'], p['ba'], p['ln_g'], p['ln_b'],
                          p['skip'])
        outs.append(_final(hp, params['pred_W'], params['pred_b'],
                           params['head_W'], params['head_b'])[:N])
    return jnp.stack(outs, 0)


# speedup vs baseline: 1.7014x; 1.1992x over previous
import math

import jax
import jax.numpy as jnp
from jax import lax
from jax.experimental import pallas as pl
from jax.experimental.pallas import tpu as pltpu
from jax.experimental.pallas import tpu_sc as plsc

N = 10000
DEG = 16
E = N * DEG
R = 2
B = 2
C_IN = 3
T_IN = 12
D_IN = C_IN * T_IN
D = 64
H = 4
DK = D // H
L = 2
T_OUT = 12

NCORE = 2
NSUB = 16
NT = NCORE * NSUB
NB = 320
NPAD = NT * NB
FLUSH = 2048
ECAP = E + 8192
CHB = 4000
CH = 128
LV = 6144
LVCAP = ((E + LV - 1) // LV) * LV

_MESH = plsc.VectorSubcoreMesh(core_axis_name="c", subcore_axis_name="s")


def _bin_body(src_hbm, dst_hbm, lp0, lp1, cnts,
              stage_s, stage_d, buf, cnt_v):
    wid = lax.axis_index("c") * NSUB + lax.axis_index("s")
    base = wid * NB
    iota = lax.iota(jnp.int32, 16)
    zero16 = jnp.zeros((16,), jnp.int32)

    def zinit(i, _):
        buf[pl.ds(i * 16, 16)] = zero16
        return 0

    lax.fori_loop(0, (FLUSH + 16) // 16, zinit, 0)

    for r, lpr in enumerate((lp0, lp1)):
        def outer(cb, carry, lpr=lpr, r=r):
            cnt, written = carry
            off = pl.multiple_of(r * E + cb * CHB, 8)
            pltpu.sync_copy(src_hbm.at[pl.ds(off, CHB)], stage_s)
            pltpu.sync_copy(dst_hbm.at[pl.ds(off, CHB)], stage_d)

            def inner(i, carry2):
                cnt, written = carry2
                d16 = stage_d[pl.ds(i * 16, 16)]
                s16 = stage_s[pl.ds(i * 16, 16)]
                dl = d16 - base
                msk = (dl >= 0) & (dl < NB)
                dlc = jnp.clip(dl, 0, NB - 1)
                packed = s16 * 512 + dlc
                key = jnp.where(msk, iota, jnp.full((16,), 16, jnp.int32))
                _, sv = lax.sort((key, packed), num_keys=1)
                buf[pl.ds(cnt, 16)] = sv
                cnt = cnt + plsc.all_reduce_population_count(msk)[0]

                def do_flush(c, w):
                    wo = pl.multiple_of(wid * ECAP + w, 8)
                    pltpu.sync_copy(buf.at[pl.ds(0, FLUSH)],
                                    lpr.at[pl.ds(wo, FLUSH)])
                    rem = buf[pl.ds(FLUSH, 16)]
                    buf[pl.ds(0, 16)] = rem
                    return c - FLUSH, w + FLUSH

                cnt, written = lax.cond(cnt >= FLUSH, do_flush,
                                        lambda c, w: (c, w), cnt, written)
                return (cnt, written)

            return lax.fori_loop(0, CHB // 16, inner, (cnt, written))

        cnt, written = lax.fori_loop(0, E // CHB, outer,
                                     (jnp.int32(0), jnp.int32(0)))
        wo = pl.multiple_of(wid * ECAP + written, 8)
        pltpu.sync_copy(buf.at[pl.ds(0, FLUSH)], lpr.at[pl.ds(wo, FLUSH)])
        cnt_v[...] = jnp.broadcast_to(written + cnt, (16,)).astype(jnp.int32)
        co = pl.multiple_of((r * NT + wid) * 16, 8)
        pltpu.sync_copy(cnt_v, cnts.at[pl.ds(co, 16)])


def _bin_edges(src, dst):
    f = pl.kernel(
        _bin_body,
        out_type=(
            jax.ShapeDtypeStruct((NT * ECAP,), jnp.int32),
            jax.ShapeDtypeStruct((NT * ECAP,), jnp.int32),
            jax.ShapeDtypeStruct((R * NT * 16,), jnp.int32),
        ),
        mesh=_MESH,
        compiler_params=pltpu.CompilerParams(needs_layout_passes=False),
        scratch_types=[
            pltpu.VMEM((CHB,), jnp.int32),
            pltpu.VMEM((CHB,), jnp.int32),
            pltpu.VMEM((FLUSH + 16,), jnp.int32),
            pltpu.VMEM((16,), jnp.int32),
        ],
    )
    return f(src.reshape(R * E), dst.reshape(R * E))


def _edge_body(q_hbm, kr0, kr1, vr0, vr1, lp0, lp1, cnts,
               agg0, agg1, lg,
               q_loc, m_buf, s_buf, acc, pk_full, tl_sup,
               src_v0, src_v1, dl_v0, dl_v1, rows0, rows1,
               cnt_v, sem0, sem1):
    wid = lax.axis_index("c") * NSUB + lax.axis_index("s")
    base = wid * NB
    iota = lax.iota(jnp.int32, 16)
    iota_h = jnp.minimum(iota, 3)
    zf = jnp.zeros((16,), jnp.float32)
    minf = jnp.full((16,), -3.0e38, jnp.float32)
    srcs = (src_v0, src_v1)
    dls = (dl_v0, dl_v1)
    rows = (rows0, rows1)
    sems = (sem0, sem1)

    pltpu.sync_copy(q_hbm.at[pl.ds(pl.multiple_of(base * D, 8), NB * D)],
                    q_loc)

    def unpack_chunk(ci, p):
        o = ci * CH

        def u(ui, _):
            pv = pk_full[pl.ds(o + ui * 16, 16)]
            sv = jnp.clip(jnp.right_shift(pv, 9), 0, N - 1)
            dv = jnp.minimum(jnp.bitwise_and(pv, 511), NB - 1)
            srcs[p][pl.ds(ui * 16, 16)] = sv
            dls[p][pl.ds(ui * 16, 16)] = dv
            return 0

        lax.fori_loop(0, CH // 16, u, 0)

    for r in range(R):
        krr = (kr0, kr1)[r]
        vrr = (vr0, vr1)[r]
        lpr = (lp0, lp1)[r]
        aggr = (agg0, agg1)[r]

        co = pl.multiple_of((r * NT + wid) * 16, 8)
        pltpu.sync_copy(cnts.at[pl.ds(co, 16)], cnt_v)
        cnt = cnt_v[...][0]

        for hh in range(4):
            m_buf[pl.ds(hh * 16, 16)] = minf

        def init_ms(i, _):
            s_buf[pl.ds(i * 16, 16)] = zf
            return 0

        lax.fori_loop(0, NB, init_ms, 0)

        def init_acc(i, _):
            acc[pl.ds(i * 16, 16)] = zf
            return 0

        lax.fori_loop(0, NB * 4, init_acc, 0)

        nsup = (cnt + LV - 1) // LV
        multi = nsup > 1

        def sup1(si, _, lpr=lpr, krr=krr, cnt=cnt, multi=multi):
            soff = si * LV
            lo = pl.multiple_of(wid * ECAP + soff, 8)
            pltpu.sync_copy(lpr.at[pl.ds(lo, LV)], pk_full)
            nval = jnp.minimum(cnt - soff, LV)
            nin = (nval + CH - 1) // CH

            @pl.when(nin > 0)
            def _():
                unpack_chunk(0, 0)
                pltpu.async_copy(krr.at[srcs[0]], rows[0], sems[0])

            def pair(ci2, _):
                for p_ in (0, 1):
                    ci = ci2 * 2 + p_

                    @pl.when(ci < nin)
                    def _(ci=ci, p_=p_):
                        @pl.when(ci + 1 < nin)
                        def _():
                            unpack_chunk(ci + 1, 1 - p_)
                            pltpu.async_copy(krr.at[srcs[1 - p_]],
                                             rows[1 - p_], sems[1 - p_])

                        pltpu.make_async_copy(krr.at[srcs[p_]], rows[p_],
                                              sems[p_]).wait()
                        coff = ci * CH

                        def sub(sc, mcar):
                            e0 = sc * 16
                            dl16 = dls[p_][pl.ds(e0, 16)]
                            dlb = dl16 * D
                            t = [zf, zf, zf, zf]
                            for c in range(D):
                                csp = jnp.full((16,), c, jnp.int32)
                                qv = plsc.load_gather(q_loc, [dlb + csp])
                                kv = plsc.load_gather(rows[p_],
                                                      [iota + e0, csp])
                                t[c // 16] = t[c // 16] + qv * kv
                            vvalid = (soff + coff + e0 + iota) < cnt
                            mcar = list(mcar)
                            for hh in range(4):
                                tl_sup[pl.ds(hh * LV + coff + e0, 16)] = t[hh]
                                mcar[hh] = jnp.maximum(
                                    mcar[hh], jnp.where(vvalid, t[hh], minf))
                            return tuple(mcar)

                        mcar = lax.fori_loop(0, CH // 16, sub,
                                             tuple(m_buf[pl.ds(hh * 16, 16)]
                                                   for hh in range(4)))
                        for hh in range(4):
                            m_buf[pl.ds(hh * 16, 16)] = mcar[hh]

                return 0

            lax.fori_loop(0, (nin + 1) // 2, pair, 0)

            @pl.when(multi)
            def _():
                go = pl.multiple_of((wid * LVCAP + soff) * 4, 8)
                pltpu.sync_copy(tl_sup, lg.at[pl.ds(go, 4 * LV)])

            return 0

        lax.fori_loop(0, nsup, sup1, 0)

        gmax = [jnp.max(m_buf[pl.ds(hh * 16, 16)]) for hh in range(4)]

        def sup2(si, _, lpr=lpr, vrr=vrr, cnt=cnt, multi=multi, gmax=gmax):
            soff = si * LV
            lo = pl.multiple_of(wid * ECAP + soff, 8)
            pltpu.sync_copy(lpr.at[pl.ds(lo, LV)], pk_full)

            @pl.when(multi)
            def _():
                go = pl.multiple_of((wid * LVCAP + soff) * 4, 8)
                pltpu.sync_copy(lg.at[pl.ds(go, 4 * LV)], tl_sup)

            nval = jnp.minimum(cnt - soff, LV)
            nin = (nval + CH - 1) // CH

            @pl.when(nin > 0)
            def _():
                unpack_chunk(0, 0)
                pltpu.async_copy(vrr.at[srcs[0]], rows[0], sems[0])

            def pair(ci2, _):
                for p_ in (0, 1):
                    ci = ci2 * 2 + p_

                    @pl.when(ci < nin)
                    def _(ci=ci, p_=p_):
                        @pl.when(ci + 1 < nin)
                        def _():
                            unpack_chunk(ci + 1, 1 - p_)
                            pltpu.async_copy(vrr.at[srcs[1 - p_]],
                                             rows[1 - p_], sems[1 - p_])

                        pltpu.make_async_copy(vrr.at[srcs[p_]], rows[p_],
                                              sems[p_]).wait()
                        coff = ci * CH

                        def sub2(sc, _):
                            e0 = sc * 16
                            dl16 = dls[p_][pl.ds(e0, 16)]
                            dlm = dl16 * 16
                            vvalid = (soff + coff + e0 + iota) < cnt
                            ev = []
                            for hh in range(4):
                                tv = tl_sup[pl.ds(hh * LV + coff + e0, 16)]
                                e_ = jnp.exp(tv - gmax[hh])
                                ev.append(jnp.where(vvalid, e_, zf))
                            for hh in range(4):
                                hsp = jnp.full((16,), hh, jnp.int32)
                                plsc.addupdate_scatter(
                                    s_buf, [dlm + hsp], ev[hh])
                            for j in range(16):
                                dlj = dl16[j]
                                for hh in range(4):
                                    e_h = ev[hh][j]
                                    vrv = rows[p_][e0 + j,
                                                   pl.ds(hh * 16, 16)]
                                    plsc.addupdate(
                                        acc.at[pl.ds(dlj * D + hh * 16, 16)],
                                        vrv * e_h)
                            return 0

                        lax.fori_loop(0, CH // 16, sub2, 0)

                return 0

            lax.fori_loop(0, (nin + 1) // 2, pair, 0)
            return 0

        lax.fori_loop(0, nsup, sup2, 0)

        def fin(i, _):
            srow = s_buf[pl.ds(i * 16, 16)]
            for hh in range(4):
                sv = srow[hh]
                sv = jnp.where(sv > 0.0, sv, 1.0)
                o = pl.ds(i * D + hh * 16, 16)
                acc[o] = acc[o] / sv
            return 0

        lax.fori_loop(0, NB, fin, 0)
        pltpu.sync_copy(
            acc, aggr.at[pl.ds(pl.multiple_of(base * D, 8), NB * D)])


def _edge_pass(qflat, kr0, kr1, vr0, vr1, lp0, lp1, cnts):
    f = pl.kernel(
        _edge_body,
        out_type=(
            jax.ShapeDtypeStruct((NPAD * D,), jnp.float32),
            jax.ShapeDtypeStruct((NPAD * D,), jnp.float32),
            jax.ShapeDtypeStruct((NT * H * LVCAP,), jnp.float32),
        ),
        mesh=_MESH,
        compiler_params=pltpu.CompilerParams(needs_layout_passes=False),
        scratch_types=[
            pltpu.VMEM((NB * D,), jnp.float32),
            pltpu.VMEM((64,), jnp.float32),
            pltpu.VMEM((NB * 16,), jnp.float32),
            pltpu.VMEM((NB * D,), jnp.float32),
            pltpu.VMEM((LV,), jnp.int32),
            pltpu.VMEM((H * LV,), jnp.float32),
            pltpu.VMEM((CH,), jnp.int32),
            pltpu.VMEM((CH,), jnp.int32),
            pltpu.VMEM((CH,), jnp.int32),
            pltpu.VMEM((CH,), jnp.int32),
            pltpu.VMEM((CH, 2 * D), jnp.float32),
            pltpu.VMEM((CH, 2 * D), jnp.float32),
            pltpu.VMEM((16,), jnp.int32),
            pltpu.SemaphoreType.DMA,
            pltpu.SemaphoreType.DMA,
        ],
    )
    a0, a1, _ = f(qflat, kr0, kr1, vr0, vr1, lp0, lp1, cnts)
    return a0.reshape(NPAD, D), a1.reshape(NPAD, D)


def _pre_body(x_ref, w_ref, b_ref, o_ref):
    h = jnp.dot(x_ref[...], w_ref[...],
                preferred_element_type=jnp.float32) + b_ref[...]
    o_ref[...] = jax.nn.gelu(h)


def _q_mm_body(h_ref, w_ref, b_ref, o_ref):
    o_ref[...] = jnp.dot(h_ref[...], w_ref[...],
                         preferred_element_type=jnp.float32) + b_ref[...]


def _q_mm(hp, w, b):
    return pl.pallas_call(
        _q_mm_body,
        out_shape=jax.ShapeDtypeStruct((NPAD, D), jnp.float32),
    )(hp, w, b.reshape(1, D))


def _pre(xp, w, b):
    return pl.pallas_call(
        _pre_body,
        out_shape=jax.ShapeDtypeStruct((NPAD, D), jnp.float32),
    )(xp, w, b.reshape(1, D))


def _qkv_body(h_ref, w_ref, b_ref, o_ref):
    res = jnp.dot(h_ref[...], w_ref[0],
                  preferred_element_type=jnp.float32) + b_ref[0]
    o_ref[0] = jnp.concatenate([res, jnp.zeros_like(res)], axis=-1)


def _qkv(hp, wstack, bstack):
    nmat = wstack.shape[0]
    bstack = bstack.reshape(nmat, 1, D)
    return pl.pallas_call(
        _qkv_body,
        grid=(nmat,),
        in_specs=[
            pl.BlockSpec((NPAD, D), lambda i: (0, 0)),
            pl.BlockSpec((1, D, D), lambda i: (i, 0, 0)),
            pl.BlockSpec((1, 1, D), lambda i: (i, 0, 0)),
        ],
        out_specs=pl.BlockSpec((1, NPAD, 2 * D), lambda i: (i, 0, 0)),
        out_shape=jax.ShapeDtypeStruct((nmat, NPAD, 2 * D), jnp.float32),
    )(hp, wstack, bstack)


def _combine_body(a0_ref, a1_ref, h_ref, wa_ref, ba_ref, g_ref, bb_ref,
                  skip_ref, o_ref):
    hs = (a0_ref[...] + a1_ref[...]) * 0.5
    trans = jnp.dot(hs, wa_ref[...],
                    preferred_element_type=jnp.float32) + ba_ref[...]
    alpha = jax.nn.sigmoid(skip_ref[0, 0])
    out = alpha * trans + (1.0 - alpha) * h_ref[...]
    mu = jnp.mean(out, axis=-1, keepdims=True)
    d = out - mu
    var = jnp.mean(d * d, axis=-1, keepdims=True)
    o_ref[...] = d * lax.rsqrt(var + 1e-5) * g_ref[...] + bb_ref[...]


def _combine(a0, a1, hp, wa, ba, g, bb, skip):
    return pl.pallas_call(
        _combine_body,
        out_shape=jax.ShapeDtypeStruct((NPAD, D), jnp.float32),
    )(a0, a1, hp, wa, ba.reshape(1, D), g.reshape(1, D), bb.reshape(1, D),
      skip.reshape(1, 1))


def _final_body(h_ref, wp_ref, bp_ref, wh_ref, bh_ref, o_ref):
    t = jnp.dot(h_ref[...], wp_ref[...],
                preferred_element_type=jnp.float32) + bp_ref[...]
    o_ref[...] = jnp.dot(t, wh_ref[...],
                         preferred_element_type=jnp.float32) + bh_ref[...]


def _final(hp, wp, bp, wh, bh):
    return pl.pallas_call(
        _final_body,
        out_shape=jax.ShapeDtypeStruct((NPAD, T_OUT), jnp.float32),
    )(hp, wp, bp.reshape(1, D), wh, bh.reshape(1, T_OUT))


def _block_diag(w):
    out = jnp.zeros((D, D), jnp.float32)
    for hh in range(H):
        out = out.at[hh * DK:(hh + 1) * DK, hh * DK:(hh + 1) * DK].set(w[hh])
    return out


def kernel(data, timestamps, params, src, dst):
    x = data.reshape(B, N, D_IN)
    xp = jnp.pad(x, ((0, 0), (0, NPAD - N), (0, 0)))

    lp0, lp1, cnts = _bin_edges(src, dst)

    layer_w = []
    for l in range(L):
        p = params['layers'][l]
        ws, bs = [], []
        for r in range(R):
            bd = _block_diag(p['w_att'][r])
            scale = jnp.repeat(p['mu'][r], DK) / math.sqrt(DK)
            ws.append((p['Wk'] @ bd) * scale[None, :])
            bs.append((p['bk'] @ bd) * scale)
        for r in range(R):
            bd = _block_diag(p['w_msg'][r])
            ws.append(p['Wv'] @ bd)
            bs.append(p['bv'] @ bd)
        layer_w.append((jnp.stack(ws), jnp.stack(bs)))

    outs = []
    for b in range(B):
        hp = _pre(xp[b], params['adapt_W'], params['adapt_b'])
        for l in range(L):
            p = params['layers'][l]
            wstack, bstack = layer_w[l]
            proj = _qkv(hp, wstack, bstack)
            q = _q_mm(hp, p['Wq'], p['bq'])
            a0, a1 = _edge_pass(q.reshape(NPAD * D), proj[0],
                                proj[1], proj[2], proj[3], lp0, lp1, cnts)
            hp = _combine(a0, a1, hp, p['Wa'], p['ba'], p['ln_g'], p['ln_b'],
                          p['skip'])
        outs.append(_final(hp, params['pred_W'], params['pred_b'],
                           params['head_W'], params['head_b'])[:N])
    return jnp.stack(outs, 0)


# speedup vs baseline: 1.7336x; 1.0189x over previous
import math

import jax
import jax.numpy as jnp
from jax import lax
from jax.experimental import pallas as pl
from jax.experimental.pallas import tpu as pltpu
from jax.experimental.pallas import tpu_sc as plsc

N = 10000
DEG = 16
E = N * DEG
R = 2
B = 2
C_IN = 3
T_IN = 12
D_IN = C_IN * T_IN
D = 64
H = 4
DK = D // H
L = 2
T_OUT = 12

NCORE = 2
NSUB = 16
NT = NCORE * NSUB
NB = 320
NPAD = NT * NB
FLUSH = 2048
ECAP = E + 8192
CHB = 16000
CH = 128
LV = 6144
LVCAP = ((E + LV - 1) // LV) * LV

_MESH = plsc.VectorSubcoreMesh(core_axis_name="c", subcore_axis_name="s")


def _bin_body(src_hbm, dst_hbm, lp0, lp1, cnts,
              stage_s, stage_d, buf, cnt_v):
    wid = lax.axis_index("c") * NSUB + lax.axis_index("s")
    base = wid * NB
    iota = lax.iota(jnp.int32, 16)
    zero16 = jnp.zeros((16,), jnp.int32)

    def zinit(i, _):
        buf[pl.ds(i * 16, 16)] = zero16
        return 0

    lax.fori_loop(0, (FLUSH + 16) // 16, zinit, 0)

    for r, lpr in enumerate((lp0, lp1)):
        def outer(cb, carry, lpr=lpr, r=r):
            cnt, written = carry
            off = pl.multiple_of(r * E + cb * CHB, 8)
            pltpu.sync_copy(src_hbm.at[pl.ds(off, CHB)], stage_s)
            pltpu.sync_copy(dst_hbm.at[pl.ds(off, CHB)], stage_d)

            def inner(i, carry2):
                cnt, written = carry2
                d16 = stage_d[pl.ds(i * 16, 16)]
                s16 = stage_s[pl.ds(i * 16, 16)]
                dl = d16 - base
                msk = (dl >= 0) & (dl < NB)
                dlc = jnp.clip(dl, 0, NB - 1)
                packed = s16 * 512 + dlc
                key = jnp.where(msk, iota, jnp.full((16,), 16, jnp.int32))
                _, sv = lax.sort((key, packed), num_keys=1)
                buf[pl.ds(cnt, 16)] = sv
                cnt = cnt + plsc.all_reduce_population_count(msk)[0]

                def do_flush(c, w):
                    wo = pl.multiple_of(wid * ECAP + w, 8)
                    pltpu.sync_copy(buf.at[pl.ds(0, FLUSH)],
                                    lpr.at[pl.ds(wo, FLUSH)])
                    rem = buf[pl.ds(FLUSH, 16)]
                    buf[pl.ds(0, 16)] = rem
                    return c - FLUSH, w + FLUSH

                cnt, written = lax.cond(cnt >= FLUSH, do_flush,
                                        lambda c, w: (c, w), cnt, written)
                return (cnt, written)

            return lax.fori_loop(0, CHB // 16, inner, (cnt, written))

        cnt, written = lax.fori_loop(0, E // CHB, outer,
                                     (jnp.int32(0), jnp.int32(0)))
        wo = pl.multiple_of(wid * ECAP + written, 8)
        pltpu.sync_copy(buf.at[pl.ds(0, FLUSH)], lpr.at[pl.ds(wo, FLUSH)])
        cnt_v[...] = jnp.broadcast_to(written + cnt, (16,)).astype(jnp.int32)
        co = pl.multiple_of((r * NT + wid) * 16, 8)
        pltpu.sync_copy(cnt_v, cnts.at[pl.ds(co, 16)])


def _bin_edges(src, dst):
    f = pl.kernel(
        _bin_body,
        out_type=(
            jax.ShapeDtypeStruct((NT * ECAP,), jnp.int32),
            jax.ShapeDtypeStruct((NT * ECAP,), jnp.int32),
            jax.ShapeDtypeStruct((R * NT * 16,), jnp.int32),
        ),
        mesh=_MESH,
        compiler_params=pltpu.CompilerParams(needs_layout_passes=False),
        scratch_types=[
            pltpu.VMEM((CHB,), jnp.int32),
            pltpu.VMEM((CHB,), jnp.int32),
            pltpu.VMEM((FLUSH + 16,), jnp.int32),
            pltpu.VMEM((16,), jnp.int32),
        ],
    )
    return f(src.reshape(R * E), dst.reshape(R * E))


def _edge_body(q_hbm, kr0, kr1, vr0, vr1, lp0, lp1, cnts,
               agg0, agg1, lg,
               q_loc, m_buf, s_buf, acc, pk_full, tl_sup,
               src_v0, src_v1, dl_v0, dl_v1, rows0, rows1,
               cnt_v, sem0, sem1):
    wid = lax.axis_index("c") * NSUB + lax.axis_index("s")
    base = wid * NB
    iota = lax.iota(jnp.int32, 16)
    iota_h = jnp.minimum(iota, 3)
    zf = jnp.zeros((16,), jnp.float32)
    minf = jnp.full((16,), -3.0e38, jnp.float32)
    srcs = (src_v0, src_v1)
    dls = (dl_v0, dl_v1)
    rows = (rows0, rows1)
    sems = (sem0, sem1)

    pltpu.sync_copy(q_hbm.at[pl.ds(pl.multiple_of(base * D, 8), NB * D)],
                    q_loc)

    def unpack_chunk(ci, p):
        o = ci * CH

        def u(ui, _):
            pv = pk_full[pl.ds(o + ui * 16, 16)]
            sv = jnp.clip(jnp.right_shift(pv, 9), 0, N - 1)
            dv = jnp.minimum(jnp.bitwise_and(pv, 511), NB - 1)
            srcs[p][pl.ds(ui * 16, 16)] = sv
            dls[p][pl.ds(ui * 16, 16)] = dv
            return 0

        lax.fori_loop(0, CH // 16, u, 0)

    for r in range(R):
        krr = (kr0, kr1)[r]
        vrr = (vr0, vr1)[r]
        lpr = (lp0, lp1)[r]
        aggr = (agg0, agg1)[r]

        co = pl.multiple_of((r * NT + wid) * 16, 8)
        pltpu.sync_copy(cnts.at[pl.ds(co, 16)], cnt_v)
        cnt = cnt_v[...][0]

        for hh in range(4):
            m_buf[pl.ds(hh * 16, 16)] = minf

        def init_ms(i, _):
            s_buf[pl.ds(i * 16, 16)] = zf
            return 0

        lax.fori_loop(0, NB, init_ms, 0)

        def init_acc(i, _):
            acc[pl.ds(i * 16, 16)] = zf
            return 0

        lax.fori_loop(0, NB * 4, init_acc, 0)

        nsup = (cnt + LV - 1) // LV
        multi = nsup > 1

        def sup1(si, _, lpr=lpr, krr=krr, cnt=cnt, multi=multi):
            soff = si * LV
            lo = pl.multiple_of(wid * ECAP + soff, 8)
            pltpu.sync_copy(lpr.at[pl.ds(lo, LV)], pk_full)
            nval = jnp.minimum(cnt - soff, LV)
            nin = (nval + CH - 1) // CH

            @pl.when(nin > 0)
            def _():
                unpack_chunk(0, 0)
                pltpu.async_copy(krr.at[srcs[0]], rows[0], sems[0])

            def pair(ci2, _):
                for p_ in (0, 1):
                    ci = ci2 * 2 + p_

                    @pl.when(ci < nin)
                    def _(ci=ci, p_=p_):
                        @pl.when(ci + 1 < nin)
                        def _():
                            unpack_chunk(ci + 1, 1 - p_)
                            pltpu.async_copy(krr.at[srcs[1 - p_]],
                                             rows[1 - p_], sems[1 - p_])

                        pltpu.make_async_copy(krr.at[srcs[p_]], rows[p_],
                                              sems[p_]).wait()
                        coff = ci * CH

                        def sub(sc, mcar):
                            e0 = sc * 16
                            dl16 = dls[p_][pl.ds(e0, 16)]
                            dlb = dl16 * D
                            t = [zf, zf, zf, zf]
                            for c in range(D):
                                csp = jnp.full((16,), c, jnp.int32)
                                qv = plsc.load_gather(q_loc, [dlb + csp])
                                kv = plsc.load_gather(rows[p_],
                                                      [iota + e0, csp])
                                t[c // 16] = t[c // 16] + qv * kv
                            vvalid = (soff + coff + e0 + iota) < cnt
                            mcar = list(mcar)
                            for hh in range(4):
                                tl_sup[pl.ds(hh * LV + coff + e0, 16)] = t[hh]
                                mcar[hh] = jnp.maximum(
                                    mcar[hh], jnp.where(vvalid, t[hh], minf))
                            return tuple(mcar)

                        mcar = lax.fori_loop(0, CH // 16, sub,
                                             tuple(m_buf[pl.ds(hh * 16, 16)]
                                                   for hh in range(4)))
                        for hh in range(4):
                            m_buf[pl.ds(hh * 16, 16)] = mcar[hh]

                return 0

            lax.fori_loop(0, (nin + 1) // 2, pair, 0)

            @pl.when(multi)
            def _():
                go = pl.multiple_of((wid * LVCAP + soff) * 4, 8)
                pltpu.sync_copy(tl_sup, lg.at[pl.ds(go, 4 * LV)])

            return 0

        lax.fori_loop(0, nsup, sup1, 0)

        gmax = [jnp.max(m_buf[pl.ds(hh * 16, 16)]) for hh in range(4)]

        def sup2(si, _, lpr=lpr, vrr=vrr, cnt=cnt, multi=multi, gmax=gmax):
            soff = si * LV
            lo = pl.multiple_of(wid * ECAP + soff, 8)
            pltpu.sync_copy(lpr.at[pl.ds(lo, LV)], pk_full)

            @pl.when(multi)
            def _():
                go = pl.multiple_of((wid * LVCAP + soff) * 4, 8)
                pltpu.sync_copy(lg.at[pl.ds(go, 4 * LV)], tl_sup)

            nval = jnp.minimum(cnt - soff, LV)
            nin = (nval + CH - 1) // CH

            @pl.when(nin > 0)
            def _():
                unpack_chunk(0, 0)
                pltpu.async_copy(vrr.at[srcs[0]], rows[0], sems[0])

            def pair(ci2, _):
                for p_ in (0, 1):
                    ci = ci2 * 2 + p_

                    @pl.when(ci < nin)
                    def _(ci=ci, p_=p_):
                        @pl.when(ci + 1 < nin)
                        def _():
                            unpack_chunk(ci + 1, 1 - p_)
                            pltpu.async_copy(vrr.at[srcs[1 - p_]],
                                             rows[1 - p_], sems[1 - p_])

                        pltpu.make_async_copy(vrr.at[srcs[p_]], rows[p_],
                                              sems[p_]).wait()
                        coff = ci * CH

                        def sub2(sc, _):
                            e0 = sc * 16
                            dl16 = dls[p_][pl.ds(e0, 16)]
                            dlm = dl16 * 16
                            vvalid = (soff + coff + e0 + iota) < cnt
                            ev = []
                            for hh in range(4):
                                tv = tl_sup[pl.ds(hh * LV + coff + e0, 16)]
                                e_ = jnp.exp(tv - gmax[hh])
                                ev.append(jnp.where(vvalid, e_, zf))
                            for hh in range(4):
                                hsp = jnp.full((16,), hh, jnp.int32)
                                plsc.addupdate_scatter(
                                    s_buf, [dlm + hsp], ev[hh])
                            for j in range(16):
                                dlj = dl16[j]
                                for hh in range(4):
                                    e_h = ev[hh][j]
                                    vrv = rows[p_][e0 + j,
                                                   pl.ds(hh * 16, 16)]
                                    plsc.addupdate(
                                        acc.at[pl.ds(dlj * D + hh * 16, 16)],
                                        vrv * e_h)
                            return 0

                        lax.fori_loop(0, CH // 16, sub2, 0)

                return 0

            lax.fori_loop(0, (nin + 1) // 2, pair, 0)
            return 0

        lax.fori_loop(0, nsup, sup2, 0)

        def fin(i, _):
            srow = s_buf[pl.ds(i * 16, 16)]
            for hh in range(4):
                sv = srow[hh]
                sv = jnp.where(sv > 0.0, sv, 1.0)
                o = pl.ds(i * D + hh * 16, 16)
                acc[o] = acc[o] / sv
            return 0

        lax.fori_loop(0, NB, fin, 0)
        pltpu.sync_copy(
            acc, aggr.at[pl.ds(pl.multiple_of(base * D, 8), NB * D)])


def _edge_pass(qflat, kr0, kr1, vr0, vr1, lp0, lp1, cnts):
    f = pl.kernel(
        _edge_body,
        out_type=(
            jax.ShapeDtypeStruct((NPAD * D,), jnp.float32),
            jax.ShapeDtypeStruct((NPAD * D,), jnp.float32),
            jax.ShapeDtypeStruct((NT * H * LVCAP,), jnp.float32),
        ),
        mesh=_MESH,
        compiler_params=pltpu.CompilerParams(needs_layout_passes=False),
        scratch_types=[
            pltpu.VMEM((NB * D,), jnp.float32),
            pltpu.VMEM((64,), jnp.float32),
            pltpu.VMEM((NB * 16,), jnp.float32),
            pltpu.VMEM((NB * D,), jnp.float32),
            pltpu.VMEM((LV,), jnp.int32),
            pltpu.VMEM((H * LV,), jnp.float32),
            pltpu.VMEM((CH,), jnp.int32),
            pltpu.VMEM((CH,), jnp.int32),
            pltpu.VMEM((CH,), jnp.int32),
            pltpu.VMEM((CH,), jnp.int32),
            pltpu.VMEM((CH, 2 * D), jnp.float32),
            pltpu.VMEM((CH, 2 * D), jnp.float32),
            pltpu.VMEM((16,), jnp.int32),
            pltpu.SemaphoreType.DMA,
            pltpu.SemaphoreType.DMA,
        ],
    )
    a0, a1, _ = f(qflat, kr0, kr1, vr0, vr1, lp0, lp1, cnts)
    return a0.reshape(NPAD, D), a1.reshape(NPAD, D)


def _pre_body(x_ref, w_ref, b_ref, o_ref):
    h = jnp.dot(x_ref[...], w_ref[...],
                preferred_element_type=jnp.float32) + b_ref[...]
    o_ref[...] = jax.nn.gelu(h)


def _q_mm_body(h_ref, w_ref, b_ref, o_ref):
    o_ref[...] = jnp.dot(h_ref[...], w_ref[...],
                         preferred_element_type=jnp.float32) + b_ref[...]


def _q_mm(hp, w, b):
    return pl.pallas_call(
        _q_mm_body,
        out_shape=jax.ShapeDtypeStruct((NPAD, D), jnp.float32),
    )(hp, w, b.reshape(1, D))


def _pre(xp, w, b):
    return pl.pallas_call(
        _pre_body,
        out_shape=jax.ShapeDtypeStruct((NPAD, D), jnp.float32),
    )(xp, w, b.reshape(1, D))


def _qkv_body(h_ref, w_ref, b_ref, o_ref):
    res = jnp.dot(h_ref[...], w_ref[0],
                  preferred_element_type=jnp.float32) + b_ref[0]
    o_ref[0] = jnp.concatenate([res, jnp.zeros_like(res)], axis=-1)


def _qkv(hp, wstack, bstack):
    nmat = wstack.shape[0]
    bstack = bstack.reshape(nmat, 1, D)
    return pl.pallas_call(
        _qkv_body,
        grid=(nmat,),
        in_specs=[
            pl.BlockSpec((NPAD, D), lambda i: (0, 0)),
            pl.BlockSpec((1, D, D), lambda i: (i, 0, 0)),
            pl.BlockSpec((1, 1, D), lambda i: (i, 0, 0)),
        ],
        out_specs=pl.BlockSpec((1, NPAD, 2 * D), lambda i: (i, 0, 0)),
        out_shape=jax.ShapeDtypeStruct((nmat, NPAD, 2 * D), jnp.float32),
    )(hp, wstack, bstack)


def _combine_body(a0_ref, a1_ref, h_ref, wa_ref, ba_ref, g_ref, bb_ref,
                  skip_ref, o_ref):
    hs = (a0_ref[...] + a1_ref[...]) * 0.5
    trans = jnp.dot(hs, wa_ref[...],
                    preferred_element_type=jnp.float32) + ba_ref[...]
    alpha = jax.nn.sigmoid(skip_ref[0, 0])
    out = alpha * trans + (1.0 - alpha) * h_ref[...]
    mu = jnp.mean(out, axis=-1, keepdims=True)
    d = out - mu
    var = jnp.mean(d * d, axis=-1, keepdims=True)
    o_ref[...] = d * lax.rsqrt(var + 1e-5) * g_ref[...] + bb_ref[...]


def _combine(a0, a1, hp, wa, ba, g, bb, skip):
    return pl.pallas_call(
        _combine_body,
        out_shape=jax.ShapeDtypeStruct((NPAD, D), jnp.float32),
    )(a0, a1, hp, wa, ba.reshape(1, D), g.reshape(1, D), bb.reshape(1, D),
      skip.reshape(1, 1))


def _final_body(h_ref, wp_ref, bp_ref, wh_ref, bh_ref, o_ref):
    t = jnp.dot(h_ref[...], wp_ref[...],
                preferred_element_type=jnp.float32) + bp_ref[...]
    o_ref[...] = jnp.dot(t, wh_ref[...],
                         preferred_element_type=jnp.float32) + bh_ref[...]


def _final(hp, wp, bp, wh, bh):
    return pl.pallas_call(
        _final_body,
        out_shape=jax.ShapeDtypeStruct((NPAD, T_OUT), jnp.float32),
    )(hp, wp, bp.reshape(1, D), wh, bh.reshape(1, T_OUT))


def _block_diag(w):
    out = jnp.zeros((D, D), jnp.float32)
    for hh in range(H):
        out = out.at[hh * DK:(hh + 1) * DK, hh * DK:(hh + 1) * DK].set(w[hh])
    return out


def kernel(data, timestamps, params, src, dst):
    x = data.reshape(B, N, D_IN)
    xp = jnp.pad(x, ((0, 0), (0, NPAD - N), (0, 0)))

    lp0, lp1, cnts = _bin_edges(src, dst)

    layer_w = []
    for l in range(L):
        p = params['layers'][l]
        ws, bs = [], []
        for r in range(R):
            bd = _block_diag(p['w_att'][r])
            scale = jnp.repeat(p['mu'][r], DK) / math.sqrt(DK)
            ws.append((p['Wk'] @ bd) * scale[None, :])
            bs.append((p['bk'] @ bd) * scale)
        for r in range(R):
            bd = _block_diag(p['w_msg'][r])
            ws.append(p['Wv'] @ bd)
            bs.append(p['bv'] @ bd)
        layer_w.append((jnp.stack(ws), jnp.stack(bs)))

    outs = []
    for b in range(B):
        hp = _pre(xp[b], params['adapt_W'], params['adapt_b'])
        for l in range(L):
            p = params['layers'][l]
            wstack, bstack = layer_w[l]
            proj = _qkv(hp, wstack, bstack)
            q = _q_mm(hp, p['Wq'], p['bq'])
            a0, a1 = _edge_pass(q.reshape(NPAD * D), proj[0],
                                proj[1], proj[2], proj[3], lp0, lp1, cnts)
            hp = _combine(a0, a1, hp, p['Wa'], p['ba'], p['ln_g'], p['ln_b'],
                          p['skip'])
        outs.append(_final(hp, params['pred_W'], params['pred_b'],
                           params['head_W'], params['head_b'])[:N])
    return jnp.stack(outs, 0)
